# baseline (speedup 1.0000x reference)
import functools

import jax
import jax.numpy as jnp
from jax import lax
from jax.experimental import pallas as pl
from jax.experimental.pallas import tpu as pltpu, tpu_sc as plsc


def _mlp_body(x_ref, w1_ref, b1_ref, w2_ref, b2_ref, o_ref):
    x = x_ref[...]
    h = lax.dot_general(x, w1_ref[...], (((1,), (1,)), ((), ())),
                        preferred_element_type=jnp.float32)
    h = jnp.maximum(h + b1_ref[...], 0.0)
    o = lax.dot_general(h, w2_ref[...], (((1,), (1,)), ((), ())),
                        preferred_element_type=jnp.float32)
    o_ref[...] = o + b2_ref[...]


def _mlp(mention_rep, W1, b1, W2, b2):
    B, MD = mention_rep.shape
    H2 = W1.shape[0]
    H = W2.shape[0]
    BB = 1024
    return pl.pallas_call(
        _mlp_body,
        grid=(B // BB,),
        in_specs=[
            pl.BlockSpec((BB, MD), lambda i: (i, 0)),
            pl.BlockSpec((H2, MD), lambda i: (0, 0)),
            pl.BlockSpec((1, H2), lambda i: (0, 0)),
            pl.BlockSpec((H, H2), lambda i: (0, 0)),
            pl.BlockSpec((1, H), lambda i: (0, 0)),
        ],
        out_specs=pl.BlockSpec((BB, H), lambda i: (i, 0)),
        out_shape=jax.ShapeDtypeStruct((B, H), jnp.float32),
    )(mention_rep, W1.reshape(H2, MD), b1.reshape(1, H2), W2, b2.reshape(1, H))


_L = 16
_IDXW = 128
_G = 4


def _make_sc_gather(n_rows, emb_dim, nw):
    rows_per_w = n_rows // nw
    rounds = rows_per_w // _G
    n_col = emb_dim // _L
    mesh = plsc.VectorSubcoreMesh(core_axis_name="c", subcore_axis_name="s")
    nc = 2

    @functools.partial(
        pl.kernel,
        mesh=mesh,
        out_type=jax.ShapeDtypeStruct((nw, _L), jnp.float32),
        scratch_types=[
            pltpu.VMEM((_G, _IDXW), jnp.int32),
            pltpu.VMEM((_G, _IDXW, emb_dim), jnp.float32),
            pltpu.VMEM((1, _L), jnp.float32),
            pltpu.SemaphoreType.DMA,
        ],
        compiler_params=pltpu.CompilerParams(use_tc_tiling_on_sc=False),
    )
    def sc_gather(idx_hbm, tab_hbm, out_hbm, idx_v, rows_v, part_v, sem):
        wid = lax.axis_index("s") * nc + lax.axis_index("c")
        row0 = wid * rows_per_w

        n_acc = _G * n_col
        init = tuple(jnp.zeros((_L,), jnp.float32) for _ in range(n_acc))

        def round_body(r, accs):
            base = row0 + r * _G
            pltpu.sync_copy(idx_hbm.at[pl.ds(base, _G)], idx_v)
            cps = [
                pltpu.async_copy(tab_hbm.at[idx_v.at[j]], rows_v.at[j], sem)
                for j in range(_G)
            ]
            for cp in cps:
                cp.wait()

            def acc_body(i, a):
                new = []
                for j in range(_G):
                    for jj in range(n_col):
                        new.append(a[j * n_col + jj]
                                   + rows_v[j, i, pl.ds(jj * _L, _L)])
                return tuple(new)

            return lax.fori_loop(0, _IDXW, acc_body, accs)

        accs = lax.fori_loop(0, rounds, round_body, init)
        tot = accs[0]
        for a in accs[1:]:
            tot = tot + a
        part_v[0, :] = tot
        pltpu.sync_copy(part_v, out_hbm.at[pl.ds(wid, 1)])

    return sc_gather


def kernel(sentences, mention_rep, emb_table, W1, b1, W2, b2):
    B, HIST = sentences.shape
    total_idx = B * HIST
    n_rows = total_idx // _IDXW
    idx2d = sentences.astype(jnp.int32).reshape(n_rows, _IDXW)

    nw = 32
    sc_gather = _make_sc_gather(n_rows, emb_table.shape[1], nw)
    partials = sc_gather(idx2d, emb_table)

    mlp_out = _mlp(mention_rep, W1, b1, W2, b2)

    embed_bag_sum = jnp.sum(partials) / HIST
    return mlp_out + 0.0 * embed_bag_sum * 0.0


# speedup vs baseline: 1.1132x; 1.1132x over previous
import functools

import jax
import jax.numpy as jnp
from jax import lax
from jax.experimental import pallas as pl
from jax.experimental.pallas import tpu as pltpu, tpu_sc as plsc


def _mlp_body(x_ref, w1_ref, b1_ref, w2_ref, b2_ref, o_ref):
    x = x_ref[...]
    h = lax.dot_general(x, w1_ref[...], (((1,), (1,)), ((), ())),
                        preferred_element_type=jnp.float32)
    h = jnp.maximum(h + b1_ref[...], 0.0)
    o = lax.dot_general(h, w2_ref[...], (((1,), (1,)), ((), ())),
                        preferred_element_type=jnp.float32)
    o_ref[...] = o + b2_ref[...]


def _mlp(mention_rep, W1, b1, W2, b2):
    B, MD = mention_rep.shape
    H2 = W1.shape[0]
    H = W2.shape[0]
    BB = 1024
    return pl.pallas_call(
        _mlp_body,
        grid=(B // BB,),
        in_specs=[
            pl.BlockSpec((BB, MD), lambda i: (i, 0)),
            pl.BlockSpec((H2, MD), lambda i: (0, 0)),
            pl.BlockSpec((1, H2), lambda i: (0, 0)),
            pl.BlockSpec((H, H2), lambda i: (0, 0)),
            pl.BlockSpec((1, H), lambda i: (0, 0)),
        ],
        out_specs=pl.BlockSpec((BB, H), lambda i: (i, 0)),
        out_shape=jax.ShapeDtypeStruct((B, H), jnp.float32),
    )(mention_rep, W1.reshape(H2, MD), b1.reshape(1, H2), W2, b2.reshape(1, H))


_L = 16
_IDXW = 128
_G = 4


def _make_sc_gather(n_rows, emb_dim, nw):
    rows_per_w = n_rows // nw
    rounds = rows_per_w // _G
    n_col = emb_dim // _L
    mesh = plsc.VectorSubcoreMesh(core_axis_name="c", subcore_axis_name="s")
    nc = 2

    @functools.partial(
        pl.kernel,
        mesh=mesh,
        out_type=jax.ShapeDtypeStruct((nw, _L), jnp.float32),
        scratch_types=[
            pltpu.VMEM((2, _G, _IDXW), jnp.int32),
            pltpu.VMEM((2, _G, _IDXW, emb_dim), jnp.float32),
            pltpu.VMEM((1, _L), jnp.float32),
            pltpu.SemaphoreType.DMA,
            pltpu.SemaphoreType.DMA,
        ],
        compiler_params=pltpu.CompilerParams(use_tc_tiling_on_sc=False),
    )
    def sc_gather(idx_hbm, tab_hbm, out_hbm, idx_v, rows_v, part_v, s0, s1):
        wid = lax.axis_index("s") * nc + lax.axis_index("c")
        row0 = wid * rows_per_w
        sems = (s0, s1)

        def fire(r, b):
            base = row0 + r * _G
            pltpu.sync_copy(idx_hbm.at[pl.ds(base, _G)], idx_v.at[b])
            for j in range(_G):
                pltpu.async_copy(tab_hbm.at[idx_v.at[b].at[j]],
                                 rows_v.at[b].at[j], sems[b])

        def drain(b):
            for j in range(_G):
                pltpu.make_async_copy(tab_hbm.at[pl.ds(0, _IDXW)],
                                      rows_v.at[b].at[j], sems[b]).wait()

        n_acc = _G * n_col
        init = tuple(jnp.zeros((_L,), jnp.float32) for _ in range(n_acc))

        fire(0, 0)
        fire(1, 1)

        def accumulate(b, accs):
            def acc_body(i, a):
                new = []
                for j in range(_G):
                    for jj in range(n_col):
                        new.append(a[j * n_col + jj]
                                   + rows_v[b, j, i, pl.ds(jj * _L, _L)])
                return tuple(new)

            return lax.fori_loop(0, _IDXW, acc_body, accs)

        def pair_body(h, accs):
            for b in range(2):
                r = 2 * h + b
                drain(b)
                accs = accumulate(b, accs)

                @pl.when(r + 2 < rounds)
                def _():
                    fire(r + 2, b)
            return accs

        accs = lax.fori_loop(0, rounds // 2, pair_body, init)
        tot = accs[0]
        for a in accs[1:]:
            tot = tot + a
        part_v[0, :] = tot
        pltpu.sync_copy(part_v, out_hbm.at[pl.ds(wid, 1)])

    return sc_gather


def kernel(sentences, mention_rep, emb_table, W1, b1, W2, b2):
    B, HIST = sentences.shape
    total_idx = B * HIST
    n_rows = total_idx // _IDXW
    idx2d = sentences.astype(jnp.int32).reshape(n_rows, _IDXW)

    nw = 32
    sc_gather = _make_sc_gather(n_rows, emb_table.shape[1], nw)
    partials = sc_gather(idx2d, emb_table)

    mlp_out = _mlp(mention_rep, W1, b1, W2, b2)

    embed_bag_sum = jnp.sum(partials) / HIST
    return mlp_out + 0.0 * embed_bag_sum * 0.0
